# Initial kernel scaffold; baseline (speedup 1.0000x reference)
import jax
import jax.numpy as jnp
from jax.experimental import pallas as pl


def kernel(labels, label_table, pos_table):
    raise NotImplementedError("write your pallas kernel here")


# speedup vs baseline: 2.3433x; 2.3433x over previous
import functools

import jax
import jax.numpy as jnp
from jax import lax
from jax.experimental import pallas as pl
from jax.experimental.pallas import tpu as pltpu
from jax.experimental.pallas import tpu_sc as plsc

B = 4096
L = 200
DIM = 64
MAXLEN = 256
NC = 2
NS = 16
NW = NC * NS
ROWS_PER_W = B // NW
LPAD = 208
SCALE = 0.7071067811865476


def _bcast_lane(vec, i):
    idx = jnp.full((16, 1), i, jnp.int32)
    return lax.gather(
        vec,
        idx,
        dimension_numbers=lax.GatherDimensionNumbers(
            offset_dims=(), collapsed_slice_dims=(0,), start_index_map=(0,)
        ),
        slice_sizes=(1,),
        mode=lax.GatherScatterMode.PROMISE_IN_BOUNDS,
    )


def _sc_body(labels_hbm, table_hbm, pos_hbm, out_hbm, lab_v, emb_v, pos_v, sem):
    wid = lax.axis_index("s") * NC + lax.axis_index("c")

    pltpu.sync_copy(pos_hbm, pos_v)

    def row_body(i, carry):
        row = wid * ROWS_PER_W + i
        base = row * L

        pltpu.sync_copy(labels_hbm.at[pl.ds(base, L)], lab_v.at[pl.ds(0, L)])
        c1 = pltpu.async_copy(
            table_hbm.at[lab_v.at[pl.ds(0, 128)]], emb_v.at[pl.ds(0, 128)], sem
        )
        c2 = pltpu.async_copy(
            table_hbm.at[lab_v.at[pl.ds(128, L - 128)]],
            emb_v.at[pl.ds(128, L - 128)],
            sem,
        )
        c1.wait()
        c2.wait()

        def grp_body(g, carry2):
            r0 = g * 16
            lab16 = lab_v[pl.ds(r0, 16)]
            mf16 = lax.sign(lab16).astype(jnp.float32)
            for i in range(16):
                m = _bcast_lane(mf16, i)
                r = r0 + i
                for j in range(DIM // 16):
                    e = emb_v[r, pl.ds(16 * j, 16)]
                    p = pos_v[r + 1, pl.ds(16 * j, 16)]
                    o = (e + p * m) * jnp.float32(SCALE)
                    emb_v[r, pl.ds(16 * j, 16)] = o
            return carry2

        lax.fori_loop(0, LPAD // 16, grp_body, 0)

        pltpu.sync_copy(emb_v.at[pl.ds(0, L)], out_hbm.at[pl.ds(base, L)])
        return carry

    lax.fori_loop(0, ROWS_PER_W, row_body, 0)


@functools.partial(jax.jit, static_argnames=())
def _run(labels_flat, label_table, pos_table):
    mesh = plsc.VectorSubcoreMesh(
        core_axis_name="c", subcore_axis_name="s", num_cores=NC, num_subcores=NS
    )
    f = pl.kernel(
        _sc_body,
        out_type=jax.ShapeDtypeStruct((B * L, DIM), jnp.float32),
        mesh=mesh,
        compiler_params=pltpu.CompilerParams(
            use_tc_tiling_on_sc=False, needs_layout_passes=False
        ),
        scratch_types=[
            pltpu.VMEM((LPAD,), jnp.int32),
            pltpu.VMEM((LPAD, DIM), jnp.float32),
            pltpu.VMEM((MAXLEN, DIM), jnp.float32),
            pltpu.SemaphoreType.DMA,
        ],
    )
    return f(labels_flat, label_table, pos_table)


def kernel(labels, label_table, pos_table):
    flat = labels.reshape(-1).astype(jnp.int32)
    out = _run(flat, label_table, pos_table)
    return out.reshape(B, L, DIM)


# speedup vs baseline: 2.8530x; 1.2175x over previous
import functools

import jax
import jax.numpy as jnp
from jax import lax
from jax.experimental import pallas as pl
from jax.experimental.pallas import tpu as pltpu
from jax.experimental.pallas import tpu_sc as plsc

B = 4096
L = 200
DIM = 64
MAXLEN = 256
NC = 2
NS = 16
NW = NC * NS
ROWS_PER_W = B // NW
LPAD = 208
NBUF = 4
SCALE = 0.7071067811865476
SPLIT = 128


def _bcast_lane(vec, i):
    idx = jnp.full((16, 1), i, jnp.int32)
    return lax.gather(
        vec,
        idx,
        dimension_numbers=lax.GatherDimensionNumbers(
            offset_dims=(), collapsed_slice_dims=(0,), start_index_map=(0,)
        ),
        slice_sizes=(1,),
        mode=lax.GatherScatterMode.PROMISE_IN_BOUNDS,
    )


def _sc_body(
    labels_hbm,
    table_hbm,
    pos_hbm,
    out_hbm,
    lab_v,
    pos_v,
    e0,
    e1,
    e2,
    e3,
    g0,
    g1,
    g2,
    g3,
    o0,
    o1,
    o2,
    o3,
):
    wid = lax.axis_index("s") * NC + lax.axis_index("c")
    wbase = wid * ROWS_PER_W * L
    embs = [e0, e1, e2, e3]
    gsems = [g0, g1, g2, g3]
    osems = [o0, o1, o2, o3]

    pltpu.sync_copy(labels_hbm.at[pl.ds(wbase, ROWS_PER_W * L)], lab_v.at[pl.ds(0, ROWS_PER_W * L)])
    pltpu.sync_copy(pos_hbm, pos_v)

    def fire_gather(c, buf):
        cb = c * L
        pltpu.async_copy(
            table_hbm.at[lab_v.at[pl.ds(cb, SPLIT)]],
            embs[buf].at[pl.ds(0, SPLIT)],
            gsems[buf],
        )
        pltpu.async_copy(
            table_hbm.at[lab_v.at[pl.ds(cb + SPLIT, L - SPLIT)]],
            embs[buf].at[pl.ds(SPLIT, L - SPLIT)],
            gsems[buf],
        )

    def wait_gather(buf):
        pltpu.make_async_copy(
            table_hbm.at[lab_v.at[pl.ds(0, SPLIT)]],
            embs[buf].at[pl.ds(0, SPLIT)],
            gsems[buf],
        ).wait()
        pltpu.make_async_copy(
            table_hbm.at[lab_v.at[pl.ds(SPLIT, L - SPLIT)]],
            embs[buf].at[pl.ds(SPLIT, L - SPLIT)],
            gsems[buf],
        ).wait()

    def fire_out(c, buf):
        pltpu.async_copy(
            embs[buf].at[pl.ds(0, L)],
            out_hbm.at[pl.ds(wbase + c * L, L)],
            osems[buf],
        )

    def wait_out(buf):
        pltpu.make_async_copy(
            embs[buf].at[pl.ds(0, L)],
            out_hbm.at[pl.ds(0, L)],
            osems[buf],
        ).wait()

    def compute(c, buf):
        emb_v = embs[buf]
        cb = c * L

        def grp_body(g, carry2):
            r0 = g * 16
            lab16 = lab_v[pl.ds(cb + r0, 16)]
            mf16 = lax.sign(lab16).astype(jnp.float32)
            for i in range(16):
                m = _bcast_lane(mf16, i)
                r = r0 + i
                for j in range(DIM // 16):
                    e = emb_v[r, pl.ds(16 * j, 16)]
                    p = pos_v[r + 1, pl.ds(16 * j, 16)]
                    o = (e + p * m) * jnp.float32(SCALE)
                    emb_v[r, pl.ds(16 * j, 16)] = o
            return carry2

        lax.fori_loop(0, LPAD // 16, grp_body, 0)

    fire_gather(0, 0)
    fire_gather(1, 1)

    def outer(k, carry):
        for j in range(NBUF):
            c = NBUF * k + j
            nbuf = (j + 2) % NBUF
            if j >= 2:
                wait_out(nbuf)
            else:

                @pl.when(k > 0)
                def _():
                    wait_out(nbuf)

            @pl.when(c + 2 < ROWS_PER_W)
            def _():
                fire_gather(c + 2, nbuf)

            wait_gather(j)
            compute(c, j)
            fire_out(c, j)
        return carry

    lax.fori_loop(0, ROWS_PER_W // NBUF, outer, 0)

    wait_out((ROWS_PER_W - 2) % NBUF)
    wait_out((ROWS_PER_W - 1) % NBUF)


@functools.partial(jax.jit, static_argnames=())
def _run(labels_flat, label_table, pos_table):
    mesh = plsc.VectorSubcoreMesh(
        core_axis_name="c", subcore_axis_name="s", num_cores=NC, num_subcores=NS
    )
    f = pl.kernel(
        _sc_body,
        out_type=jax.ShapeDtypeStruct((B * L, DIM), jnp.float32),
        mesh=mesh,
        compiler_params=pltpu.CompilerParams(
            use_tc_tiling_on_sc=False, needs_layout_passes=False
        ),
        scratch_types=(
            [
                pltpu.VMEM((ROWS_PER_W * L + 16,), jnp.int32),
                pltpu.VMEM((MAXLEN, DIM), jnp.float32),
            ]
            + [pltpu.VMEM((LPAD, DIM), jnp.float32) for _ in range(NBUF)]
            + [pltpu.SemaphoreType.DMA for _ in range(2 * NBUF)]
        ),
    )
    return f(labels_flat, label_table, pos_table)


def kernel(labels, label_table, pos_table):
    flat = labels.reshape(-1).astype(jnp.int32)
    out = _run(flat, label_table, pos_table)
    return out.reshape(B, L, DIM)
